# Initial kernel scaffold; baseline (speedup 1.0000x reference)
import jax
import jax.numpy as jnp
from jax.experimental import pallas as pl


def kernel(x, e1_w, e1_b, e1_g, e1_be, e2_w, e2_b, e2_g, e2_be, d1_w, d1_b, d1_g, d1_be, d2_w, d2_b, d2_g, d2_be, d3_w, d3_b):
    raise NotImplementedError("write your pallas kernel here")


# speedup vs baseline: 21.2838x; 21.2838x over previous
import functools

import jax
import jax.numpy as jnp
from jax import lax
from jax.experimental import pallas as pl
from jax.experimental.pallas import tpu as pltpu

_BN_EPS = 1e-5
_C = 32
_VMEM = 48 * 1024 * 1024


def _stats(y):
    s = jnp.sum(y, axis=0, keepdims=True)
    q = jnp.sum(y * y, axis=0, keepdims=True)
    return s, q


def _e1_kernel(p_ref, w_ref, b_ref, o_ref, s_ref, q_ref):
    y = jnp.dot(p_ref[...], w_ref[...], preferred_element_type=jnp.float32)
    y = jnp.maximum(y + b_ref[...], 0.0)
    s, q = _stats(y)
    s_ref[0] = s
    q_ref[0] = q
    o_ref[...] = y.astype(jnp.bfloat16)


def _e2_kernel(a_ref, sc_ref, sh_ref, w_ref, b_ref, o_ref, s_ref, q_ref):
    ib = a_ref.shape[0]
    hp2 = a_ref.shape[3]
    ho = hp2 - 1
    m = ib * ho * ho
    sc = sc_ref[...]
    sh = sh_ref[...]
    planes = {}
    for pi in range(2):
        for pj in range(2):
            core = a_ref[:, pi, pj, 1 - pi:hp2 - pi, 1 - pj:hp2 - pj, :]
            h = core.astype(jnp.float32) * sc + sh
            h = h.astype(jnp.bfloat16)
            planes[(pi, pj)] = jnp.pad(
                h, ((0, 0), (1 - pi, pi), (1 - pj, pj), (0, 0)))
    acc = jnp.zeros((m, _C), jnp.float32)
    for di in range(3):
        for dj in range(3):
            pi, ai = di % 2, di // 2
            pj, aj = dj % 2, dj // 2
            win = planes[(pi, pj)][:, ai:ai + ho, aj:aj + ho, :]
            wt = w_ref[(di * 3 + dj) * _C:(di * 3 + dj + 1) * _C, :]
            acc += jnp.dot(win.reshape(m, _C), wt,
                           preferred_element_type=jnp.float32)
    y = jnp.maximum(acc + b_ref[...], 0.0)
    s, q = _stats(y)
    s_ref[0] = s
    q_ref[0] = q
    o_ref[...] = y.astype(jnp.bfloat16).reshape(ib, ho, ho, _C)


def _convt2x_kernel(quant, a_ref, sc_ref, sh_ref, w_ref, b_ref,
                    o_ref, s_ref, q_ref):
    ib, H, W, C = a_ref.shape
    z = a_ref[...].astype(jnp.float32) * sc_ref[...] + sh_ref[...]
    if quant:
        z = jnp.clip(jnp.tanh(z), 0.0, 1.0)
        z = jnp.round(z * 31.0) * (1.0 / 31.0)
    zp = jnp.pad(z.astype(jnp.bfloat16), ((0, 0), (0, 1), (0, 1), (0, 0)))
    m = ib * H * W
    acc = jnp.zeros((m, 4 * C), jnp.float32)
    for a in range(2):
        for b in range(2):
            win = zp[:, a:a + H, b:b + W, :]
            wt = w_ref[(a * 2 + b) * C:(a * 2 + b + 1) * C, :]
            acc += jnp.dot(win.reshape(m, C), wt,
                           preferred_element_type=jnp.float32)
    y = jnp.maximum(acc + b_ref[...], 0.0)
    s, q = _stats(y)
    s_ref[0] = s
    q_ref[0] = q
    yb = y.astype(jnp.bfloat16).reshape(ib, H, W, 2, 2, C)
    yb = jnp.transpose(yb, (0, 1, 3, 2, 4, 5)).reshape(ib, 2 * H, 2 * W, C)
    o_ref[...] = yb


def _d3_kernel(a_ref, sc_ref, sh_ref, w_ref, b_ref, o_ref):
    ib, H, W, C = a_ref.shape
    d = a_ref[...].astype(jnp.float32) * sc_ref[...] + sh_ref[...]
    dp = jnp.pad(d.astype(jnp.bfloat16), ((0, 0), (1, 1), (1, 1), (0, 0)))
    m = ib * H * W
    acc = jnp.zeros((m, 8), jnp.float32)
    for di in range(3):
        for dj in range(3):
            win = dp[:, di:di + H, dj:dj + W, :]
            wt = w_ref[(di * 3 + dj) * C:(di * 3 + dj + 1) * C, :]
            acc += jnp.dot(win.reshape(m, C), wt,
                           preferred_element_type=jnp.float32)
    y = jax.nn.sigmoid(acc + b_ref[...])
    o_ref[...] = y[:, 0:1].reshape(ib, H, W)


def _bn_scale_shift(s, q, count, gamma, beta, groups=1):
    s = jnp.sum(s, axis=0)
    q = jnp.sum(q, axis=0)
    if groups > 1:
        s = jnp.sum(s.reshape(groups, _C), axis=0, keepdims=True)
        q = jnp.sum(q.reshape(groups, _C), axis=0, keepdims=True)
        count *= groups
    inv = 1.0 / float(count)
    mean = s * inv
    var = jnp.maximum(q * inv - mean * mean, 0.0)
    scale = gamma * lax.rsqrt(var + _BN_EPS)
    shift = beta - mean * scale
    return scale, shift


def _im2col_e1(x3):
    N, H, _ = x3.shape
    ho = H // 2
    xp = jnp.pad(x3.astype(jnp.bfloat16), ((0, 0), (1, 1), (1, 1)))
    cols = [lax.slice(xp, (0, di, dj), (N, di + H - 1, dj + H - 1), (1, 2, 2))
            for di in range(3) for dj in range(3)]
    p = jnp.stack(cols, axis=-1).reshape(N * ho * ho, 9)
    return jnp.pad(p, ((0, 0), (0, 7)))


def _parity_split(a1):
    N, H, _, C = a1.shape
    hp = H + 2
    ap = jnp.pad(a1, ((0, 0), (1, 1), (1, 1), (0, 0)))
    ap = ap.reshape(N, hp // 2, 2, hp // 2, 2, C)
    return jnp.transpose(ap, (0, 2, 4, 1, 3, 5))


def _grid_call(body, nb, in_arrs, in_blocks, out_shapes, out_blocks):
    def spec(blk, lead):
        nd = len(blk)
        if lead:
            return pl.BlockSpec(blk, lambda i, nd=nd: (i,) + (0,) * (nd - 1))
        return pl.BlockSpec(blk, lambda i, nd=nd: (0,) * nd)

    return pl.pallas_call(
        body,
        out_shape=tuple(out_shapes),
        grid=(nb,),
        in_specs=[spec(blk, lead) for blk, lead in in_blocks],
        out_specs=tuple(spec(blk, lead) for blk, lead in out_blocks),
        compiler_params=pltpu.CompilerParams(
            dimension_semantics=("parallel",),
            vmem_limit_bytes=_VMEM),
    )(*in_arrs)


def kernel(x, e1_w, e1_b, e1_g, e1_be, e2_w, e2_b, e2_g, e2_be,
           d1_w, d1_b, d1_g, d1_be, d2_w, d2_b, d2_g, d2_be, d3_w, d3_b):
    N = x.shape[0]
    H = x.shape[2]
    f32 = jnp.float32
    bf16 = jnp.bfloat16

    e1_w32, e1_b32 = e1_w[:, :_C], e1_b[:, :_C]
    e2_w32, e2_b32 = e2_w[:, :_C], e2_b[:, :_C]
    e1_g32, e1_be32 = e1_g[:, :_C], e1_be[:, :_C]
    e2_g32, e2_be32 = e2_g[:, :_C], e2_be[:, :_C]

    h1 = H // 2
    M1 = N * h1 * h1
    tm1 = min(8192, M1)
    nb1 = M1 // tm1
    p1 = _im2col_e1(x.reshape(N, H, H))
    a1, s1, q1 = _grid_call(
        _e1_kernel, nb1,
        (p1, e1_w32, e1_b32),
        [((tm1, 16), True), ((16, _C), False), ((1, _C), False)],
        (jax.ShapeDtypeStruct((M1, _C), bf16),
         jax.ShapeDtypeStruct((nb1, 1, _C), f32),
         jax.ShapeDtypeStruct((nb1, 1, _C), f32)),
        (((tm1, _C), True), ((1, 1, _C), True), ((1, 1, _C), True)))
    sc1, sh1 = _bn_scale_shift(s1, q1, M1, e1_g32, e1_be32)
    a1s = _parity_split(a1.reshape(N, h1, h1, _C))

    ib2 = min(8, N)
    nb2 = N // ib2
    h2 = h1 // 2
    hp2 = h2 + 1
    a2, s2, q2 = _grid_call(
        _e2_kernel, nb2,
        (a1s, sc1, sh1, e2_w32, e2_b32),
        [((ib2, 2, 2, hp2, hp2, _C), True), ((1, _C), False), ((1, _C), False),
         ((9 * _C, _C), False), ((1, _C), False)],
        (jax.ShapeDtypeStruct((N, h2, h2, _C), bf16),
         jax.ShapeDtypeStruct((nb2, 1, _C), f32),
         jax.ShapeDtypeStruct((nb2, 1, _C), f32)),
        (((ib2, h2, h2, _C), True), ((1, 1, _C), True), ((1, 1, _C), True)))
    sc2, sh2 = _bn_scale_shift(s2, q2, N * h2 * h2, e2_g32, e2_be32)

    ib3 = min(8, N)
    nb3 = N // ib3
    a3, s3, q3 = _grid_call(
        functools.partial(_convt2x_kernel, True), nb3,
        (a2, sc2, sh2, d1_w, d1_b),
        [((ib3, h2, h2, _C), True), ((1, _C), False), ((1, _C), False),
         ((4 * _C, 4 * _C), False), ((1, 4 * _C), False)],
        (jax.ShapeDtypeStruct((N, 2 * h2, 2 * h2, _C), bf16),
         jax.ShapeDtypeStruct((nb3, 1, 4 * _C), f32),
         jax.ShapeDtypeStruct((nb3, 1, 4 * _C), f32)),
        (((ib3, 2 * h2, 2 * h2, _C), True), ((1, 1, 4 * _C), True),
         ((1, 1, 4 * _C), True)))
    sc3, sh3 = _bn_scale_shift(s3, q3, N * h2 * h2, d1_g, d1_be, groups=4)

    ib4 = 2
    nb4 = N // ib4
    h4 = 2 * h2
    a4, s4, q4 = _grid_call(
        functools.partial(_convt2x_kernel, False), nb4,
        (a3, sc3, sh3, d2_w, d2_b),
        [((ib4, h4, h4, _C), True), ((1, _C), False), ((1, _C), False),
         ((4 * _C, 4 * _C), False), ((1, 4 * _C), False)],
        (jax.ShapeDtypeStruct((N, 2 * h4, 2 * h4, _C), bf16),
         jax.ShapeDtypeStruct((nb4, 1, 4 * _C), f32),
         jax.ShapeDtypeStruct((nb4, 1, 4 * _C), f32)),
        (((ib4, 2 * h4, 2 * h4, _C), True), ((1, 1, 4 * _C), True),
         ((1, 1, 4 * _C), True)))
    sc4, sh4 = _bn_scale_shift(s4, q4, N * h4 * h4, d2_g, d2_be, groups=4)

    y = _grid_call(
        _d3_kernel, N,
        (a4, sc4, sh4, d3_w, d3_b),
        [((1, H, H, _C), True), ((1, _C), False), ((1, _C), False),
         ((9 * _C, 8), False), ((1, 8), False)],
        (jax.ShapeDtypeStruct((N, H, H), f32),),
        (((1, H, H), True),))[0]
    return y.reshape(N, 1, H, H)
